# baseline (speedup 1.0000x reference)
import functools

import jax
import jax.numpy as jnp
from jax import lax
from jax.experimental import pallas as pl
from jax.experimental.pallas import tpu as pltpu
from jax.experimental.pallas import tpu_sc as plsc

SEQ = 512
TGT = 128
LOC_EMB = 128
TIM_SIZE = 48
TIM_EMB = 32
UID_SIZE = 1000
UID_EMB = 64
DIM = LOC_EMB + TIM_EMB
OUTD = DIM + UID_EMB
VOCAB = 100000
TILE = 6144
NT = -(-VOCAB // TILE)
NTA = 8
NTB = NT - NTA

_NC = 2
_ROWS_PER_WORKER = 8


def _sc_gather_rows(table, idx):
    mesh = plsc.VectorSubcoreMesh(core_axis_name="c", subcore_axis_name="s")

    @functools.partial(
        pl.kernel,
        mesh=mesh,
        out_type=jax.ShapeDtypeStruct((TGT, LOC_EMB), jnp.float32),
        scratch_types=[
            pltpu.VMEM((_ROWS_PER_WORKER,), jnp.int32),
            pltpu.VMEM((_ROWS_PER_WORKER, LOC_EMB), jnp.float32),
            pltpu.SemaphoreType.DMA,
        ],
    )
    def gather_kernel(table_hbm, idx_hbm, out_hbm, idx_v, rows_v, sem):
        wid = lax.axis_index("s") * _NC + lax.axis_index("c")

        @pl.when(wid < TGT // _ROWS_PER_WORKER)
        def _():
            base = wid * _ROWS_PER_WORKER
            pltpu.sync_copy(idx_hbm.at[pl.ds(base, _ROWS_PER_WORKER)], idx_v)
            pltpu.async_copy(table_hbm.at[idx_v], rows_v, sem).wait()
            pltpu.sync_copy(rows_v, out_hbm.at[pl.ds(base, _ROWS_PER_WORKER)])

    return gather_kernel(table, idx)


def _fused_body(lx_ref, tim_ref, uid_ref, etw_ref, euw_ref,
                whr_ref, bhr_ref, whz_ref, bhz_ref, whh_ref, bhh_ref,
                wfa_ref, bfa_ref, wfb_ref, bfb_ref,
                out_ref, x_s, ybuf, m_s, s_s):
    p = pl.program_id(0)
    j = pl.program_id(1)

    @pl.when((p == 0) & (j == 0))
    def _prologue():
        lx = lx_ref[...]
        tids = tim_ref[...]
        oh_t = (lax.broadcasted_iota(jnp.int32, (TGT, TIM_SIZE), 1)
                == tids).astype(jnp.float32)
        tx = jnp.dot(oh_t, etw_ref[...], preferred_element_type=jnp.float32)
        x = jnp.concatenate([lx, tx], axis=1)
        e = jax.lax.dot_general(x, x, (((1,), (1,)), ((), ())),
                                preferred_element_type=jnp.float32)
        e = e - jnp.max(e, axis=1, keepdims=True)
        w = jnp.exp(e)
        w = w / jnp.sum(w, axis=1, keepdims=True)
        h = jnp.dot(w, x, preferred_element_type=jnp.float32)
        for _ in range(4):
            r = jax.nn.sigmoid(
                lax.dot_general(h, whr_ref[...], (((1,), (1,)), ((), ())),
                                preferred_element_type=jnp.float32)
                + bhr_ref[...])
            z = jax.nn.sigmoid(
                lax.dot_general(h, whz_ref[...], (((1,), (1,)), ((), ())),
                                preferred_element_type=jnp.float32)
                + bhz_ref[...])
            u = jnp.tanh(
                lax.dot_general(r * h, whh_ref[...], (((1,), (1,)), ((), ())),
                                preferred_element_type=jnp.float32)
                + bhh_ref[...])
            h = h + 0.25 * (1.0 - z) * (u - h)
        oh_u = (lax.broadcasted_iota(jnp.int32, (TGT, UID_SIZE), 1)
                == uid_ref[...]).astype(jnp.float32)
        ue = jnp.dot(oh_u, euw_ref[...], preferred_element_type=jnp.float32)
        x224 = jnp.concatenate([h, ue], axis=1)
        x_s[...] = x224.astype(jnp.bfloat16)
        m_s[...] = jnp.full((TGT, 1), -jnp.inf, jnp.float32)
        s_s[...] = jnp.zeros((TGT, 1), jnp.float32)

    def _update(ym):
        m_old = m_s[...]
        m_new = jnp.maximum(m_old, jnp.max(ym, axis=1, keepdims=True))
        s_s[...] = (s_s[...] * jnp.exp(m_old - m_new)
                    + jnp.sum(jnp.exp(ym - m_new), axis=1, keepdims=True))
        m_s[...] = m_new

    def _logits(w_ref, b_ref):
        return lax.dot_general(x_s[...], w_ref[...].astype(jnp.bfloat16),
                               (((1,), (1,)), ((), ())),
                               preferred_element_type=jnp.float32) + b_ref[...]

    @pl.when((p == 0) & (j < NTA))
    def _acc_a():
        y = _logits(wfa_ref, bfa_ref)
        ybuf[j] = y.astype(jnp.bfloat16)
        _update(y)

    @pl.when((p == 0) & (j < NTB - 1))
    def _acc_b_full():
        y = _logits(wfb_ref, bfb_ref)
        ybuf[NTA + j] = y.astype(jnp.bfloat16)
        _update(y)

    @pl.when((p == 0) & (j == NTB - 1))
    def _acc_b_ragged():
        y = _logits(wfb_ref, bfb_ref)
        ybuf[NTA + j] = y.astype(jnp.bfloat16)
        col = lax.broadcasted_iota(jnp.int32, (TGT, TILE), 1)
        _update(jnp.where(col < VOCAB - (NT - 1) * TILE, y, -jnp.inf))

    @pl.when(p == 1)
    def _writeout():
        lse = m_s[...] + jnp.log(s_s[...])
        out_ref[...] = ybuf[j].astype(jnp.float32) - lse


def kernel(loc, tim, history_loc, history_tim, history_count, uid, target_len,
           emb_loc_w, emb_tim_w, emb_uid_w, W_hr, b_hr, W_hz, b_hz,
           W_hh, b_hh, W_f, b_f):
    del history_loc, history_tim, history_count, target_len

    loc_tail = loc[SEQ - TGT:, 0]
    tim_tail = tim[SEQ - TGT:, :]
    uid2 = uid.reshape(1, 1)

    lx = _sc_gather_rows(emb_loc_w, loc_tail)

    const = lambda *_: (0, 0)
    grid = (2, NT)
    out = pl.pallas_call(
        _fused_body,
        grid=grid,
        in_specs=[
            pl.BlockSpec((TGT, LOC_EMB), const),
            pl.BlockSpec((TGT, 1), const),
            pl.BlockSpec((1, 1), const),
            pl.BlockSpec((TIM_SIZE, TIM_EMB), const),
            pl.BlockSpec((UID_SIZE, UID_EMB), const),
            pl.BlockSpec((DIM, DIM), const),
            pl.BlockSpec((1, DIM), const),
            pl.BlockSpec((DIM, DIM), const),
            pl.BlockSpec((1, DIM), const),
            pl.BlockSpec((DIM, DIM), const),
            pl.BlockSpec((1, DIM), const),
            pl.BlockSpec(
                (TILE, OUTD),
                lambda p, j: (jnp.minimum(j, NTA - 1) * (1 - p)
                              + (NTA - 1) * p, 0)),
            pl.BlockSpec(
                (1, TILE),
                lambda p, j: (0, jnp.minimum(j, NTA - 1) * (1 - p)
                              + (NTA - 1) * p)),
            pl.BlockSpec(
                (TILE, OUTD),
                lambda p, j: ((NTA + jnp.minimum(j, NTB - 1)) * (1 - p)
                              + (NT - 1) * p, 0)),
            pl.BlockSpec(
                (1, TILE),
                lambda p, j: (0, (NTA + jnp.minimum(j, NTB - 1)) * (1 - p)
                              + (NT - 1) * p)),
        ],
        out_specs=pl.BlockSpec((TGT, TILE), lambda p, j: (0, j * p)),
        out_shape=jax.ShapeDtypeStruct((TGT, VOCAB), jnp.float32),
        scratch_shapes=[
            pltpu.VMEM((TGT, OUTD), jnp.bfloat16),
            pltpu.VMEM((NT, TGT, TILE), jnp.bfloat16),
            pltpu.VMEM((TGT, 1), jnp.float32),
            pltpu.VMEM((TGT, 1), jnp.float32),
        ],
        compiler_params=pltpu.CompilerParams(
            dimension_semantics=("arbitrary", "arbitrary"),
            vmem_limit_bytes=100 * 1024 * 1024,
        ),
    )(lx, tim_tail, uid2, emb_tim_w, emb_uid_w,
      W_hr, b_hr.reshape(1, DIM), W_hz, b_hz.reshape(1, DIM),
      W_hh, b_hh.reshape(1, DIM),
      W_f, b_f.reshape(1, VOCAB), W_f, b_f.reshape(1, VOCAB))
    return out


# speedup vs baseline: 1.0030x; 1.0030x over previous
import functools

import jax
import jax.numpy as jnp
from jax import lax
from jax.experimental import pallas as pl
from jax.experimental.pallas import tpu as pltpu
from jax.experimental.pallas import tpu_sc as plsc

SEQ = 512
TGT = 128
LOC_EMB = 128
TIM_SIZE = 48
TIM_EMB = 32
UID_SIZE = 1000
UID_EMB = 64
DIM = LOC_EMB + TIM_EMB
OUTD = DIM + UID_EMB
VOCAB = 100000
TILE = 10240
NT = -(-VOCAB // TILE)

_NC = 2
_ROWS_PER_WORKER = 8


def _sc_gather_rows(table, idx):
    mesh = plsc.VectorSubcoreMesh(core_axis_name="c", subcore_axis_name="s")

    @functools.partial(
        pl.kernel,
        mesh=mesh,
        out_type=jax.ShapeDtypeStruct((TGT, LOC_EMB), jnp.float32),
        scratch_types=[
            pltpu.VMEM((_ROWS_PER_WORKER,), jnp.int32),
            pltpu.VMEM((_ROWS_PER_WORKER, LOC_EMB), jnp.float32),
            pltpu.SemaphoreType.DMA,
        ],
    )
    def gather_kernel(table_hbm, idx_hbm, out_hbm, idx_v, rows_v, sem):
        wid = lax.axis_index("s") * _NC + lax.axis_index("c")

        @pl.when(wid < TGT // _ROWS_PER_WORKER)
        def _():
            base = wid * _ROWS_PER_WORKER
            pltpu.sync_copy(idx_hbm.at[pl.ds(base, _ROWS_PER_WORKER)], idx_v)
            pltpu.async_copy(table_hbm.at[idx_v], rows_v, sem).wait()
            pltpu.sync_copy(rows_v, out_hbm.at[pl.ds(base, _ROWS_PER_WORKER)])

    return gather_kernel(table, idx)


def _fused_body(lx_ref, tim_ref, uid_ref, etw_ref, euw_ref,
                whr_ref, bhr_ref, whz_ref, bhz_ref, whh_ref, bhh_ref,
                wf_ref, bf_ref, out_ref, x_s, ybuf, m_s, s_s):
    p = pl.program_id(0)
    j = pl.program_id(1)

    @pl.when((p == 0) & (j == 0))
    def _prologue():
        lx = lx_ref[...]
        tids = tim_ref[...]
        oh_t = (lax.broadcasted_iota(jnp.int32, (TGT, TIM_SIZE), 1)
                == tids).astype(jnp.float32)
        tx = jnp.dot(oh_t, etw_ref[...], preferred_element_type=jnp.float32)
        x = jnp.concatenate([lx, tx], axis=1)
        e = jax.lax.dot_general(x, x, (((1,), (1,)), ((), ())),
                                preferred_element_type=jnp.float32)
        e = e - jnp.max(e, axis=1, keepdims=True)
        w = jnp.exp(e)
        w = w / jnp.sum(w, axis=1, keepdims=True)
        h = jnp.dot(w, x, preferred_element_type=jnp.float32)
        for _ in range(4):
            r = jax.nn.sigmoid(
                lax.dot_general(h, whr_ref[...], (((1,), (1,)), ((), ())),
                                preferred_element_type=jnp.float32)
                + bhr_ref[...])
            z = jax.nn.sigmoid(
                lax.dot_general(h, whz_ref[...], (((1,), (1,)), ((), ())),
                                preferred_element_type=jnp.float32)
                + bhz_ref[...])
            u = jnp.tanh(
                lax.dot_general(r * h, whh_ref[...], (((1,), (1,)), ((), ())),
                                preferred_element_type=jnp.float32)
                + bhh_ref[...])
            h = h + 0.25 * (1.0 - z) * (u - h)
        oh_u = (lax.broadcasted_iota(jnp.int32, (TGT, UID_SIZE), 1)
                == uid_ref[...]).astype(jnp.float32)
        ue = jnp.dot(oh_u, euw_ref[...], preferred_element_type=jnp.float32)
        x224 = jnp.concatenate([h, ue], axis=1)
        x_s[...] = x224.astype(jnp.bfloat16)
        m_s[...] = jnp.full((TGT, 1), -jnp.inf, jnp.float32)
        s_s[...] = jnp.zeros((TGT, 1), jnp.float32)

    @pl.when(p == 0)
    def _accumulate():
        y = lax.dot_general(x_s[...], wf_ref[...].astype(jnp.bfloat16),
                            (((1,), (1,)), ((), ())),
                            preferred_element_type=jnp.float32) + bf_ref[...]
        ybuf[j] = y.astype(jnp.bfloat16)

        def _update(ym):
            m_old = m_s[...]
            m_new = jnp.maximum(m_old, jnp.max(ym, axis=1, keepdims=True))
            s_s[...] = (s_s[...] * jnp.exp(m_old - m_new)
                        + jnp.sum(jnp.exp(ym - m_new), axis=1, keepdims=True))
            m_s[...] = m_new

        @pl.when(j < NT - 1)
        def _full():
            _update(y)

        @pl.when(j == NT - 1)
        def _ragged():
            col = lax.broadcasted_iota(jnp.int32, (TGT, TILE), 1)
            _update(jnp.where(col < VOCAB - (NT - 1) * TILE, y, -jnp.inf))

    @pl.when(p == 1)
    def _writeout():
        lse = m_s[...] + jnp.log(s_s[...])
        out_ref[...] = ybuf[j].astype(jnp.float32) - lse


def kernel(loc, tim, history_loc, history_tim, history_count, uid, target_len,
           emb_loc_w, emb_tim_w, emb_uid_w, W_hr, b_hr, W_hz, b_hz,
           W_hh, b_hh, W_f, b_f):
    del history_loc, history_tim, history_count, target_len

    loc_tail = loc[SEQ - TGT:, 0]
    tim_tail = tim[SEQ - TGT:, :]
    uid2 = uid.reshape(1, 1)

    lx = _sc_gather_rows(emb_loc_w, loc_tail)

    const = lambda *_: (0, 0)
    grid = (2, NT)
    out = pl.pallas_call(
        _fused_body,
        grid=grid,
        in_specs=[
            pl.BlockSpec((TGT, LOC_EMB), const),
            pl.BlockSpec((TGT, 1), const),
            pl.BlockSpec((1, 1), const),
            pl.BlockSpec((TIM_SIZE, TIM_EMB), const),
            pl.BlockSpec((UID_SIZE, UID_EMB), const),
            pl.BlockSpec((DIM, DIM), const),
            pl.BlockSpec((1, DIM), const),
            pl.BlockSpec((DIM, DIM), const),
            pl.BlockSpec((1, DIM), const),
            pl.BlockSpec((DIM, DIM), const),
            pl.BlockSpec((1, DIM), const),
            pl.BlockSpec((TILE, OUTD),
                         lambda p, j: (j * (1 - p) + p * (NT - 1), 0)),
            pl.BlockSpec((1, TILE),
                         lambda p, j: (0, j * (1 - p) + p * (NT - 1))),
        ],
        out_specs=pl.BlockSpec((TGT, TILE), lambda p, j: (0, j * p)),
        out_shape=jax.ShapeDtypeStruct((TGT, VOCAB), jnp.float32),
        scratch_shapes=[
            pltpu.VMEM((TGT, OUTD), jnp.bfloat16),
            pltpu.VMEM((NT, TGT, TILE), jnp.bfloat16),
            pltpu.VMEM((TGT, 1), jnp.float32),
            pltpu.VMEM((TGT, 1), jnp.float32),
        ],
        compiler_params=pltpu.CompilerParams(
            dimension_semantics=("arbitrary", "arbitrary"),
            vmem_limit_bytes=100 * 1024 * 1024,
        ),
    )(lx, tim_tail, uid2, emb_tim_w, emb_uid_w,
      W_hr, b_hr.reshape(1, DIM), W_hz, b_hz.reshape(1, DIM),
      W_hh, b_hh.reshape(1, DIM), W_f, b_f.reshape(1, VOCAB))
    return out
